# Initial kernel scaffold; baseline (speedup 1.0000x reference)
import functools

import jax
import jax.numpy as jnp
from jax import lax
from jax.experimental import pallas as pl
from jax.experimental.pallas import tpu as pltpu
from jax.experimental.pallas import tpu_sc as plsc

N = 10000
NB = 16
AF = 256
BF = 16
H = 256
M = 250

_EDGES = N * NB
_NW = 32
_PER_W = _EDGES // _NW
_CH = 128
_NFULL = _PER_W // _CH
_TAIL = _PER_W - _NFULL * _CH


def _sc_gather(table, idx, d, dtype):
    mesh = plsc.VectorSubcoreMesh(core_axis_name="c", subcore_axis_name="s")

    @functools.partial(
        pl.kernel,
        mesh=mesh,
        out_type=jax.ShapeDtypeStruct((_EDGES, d), dtype),
        scratch_types=[
            pltpu.VMEM((_CH,), jnp.int32),
            pltpu.VMEM((_CH, d), dtype),
            pltpu.VMEM((_TAIL,), jnp.int32),
            pltpu.VMEM((_TAIL, d), dtype),
            pltpu.SemaphoreType.DMA,
        ],
    )
    def gather_kernel(table_hbm, idx_hbm, out_hbm, idx_v, rows_v, idx_t, rows_t, sem):
        wid = lax.axis_index("s") * 2 + lax.axis_index("c")
        base = wid * _PER_W

        def body(c, carry):
            off = base + c * _CH
            pltpu.sync_copy(idx_hbm.at[pl.ds(off, _CH)], idx_v)
            pltpu.async_copy(table_hbm.at[idx_v], rows_v, sem).wait()
            pltpu.sync_copy(rows_v, out_hbm.at[pl.ds(off, _CH)])
            return carry

        lax.fori_loop(0, _NFULL, body, 0)
        off = base + _NFULL * _CH
        pltpu.sync_copy(idx_hbm.at[pl.ds(off, _TAIL)], idx_t)
        pltpu.async_copy(table_hbm.at[idx_t], rows_t, sem).wait()
        pltpu.sync_copy(rows_t, out_hbm.at[pl.ds(off, _TAIL)])

    return gather_kernel(table, idx)


def _lin(x, wT, b):
    A = 2000
    K = x.shape[1]

    def body(x_ref, w_ref, b_ref, o_ref):
        o_ref[...] = (
            jnp.dot(x_ref[...], w_ref[...], preferred_element_type=jnp.float32)
            + b_ref[...]
        )

    return pl.pallas_call(
        body,
        grid=(N // A,),
        in_specs=[
            pl.BlockSpec((A, K), lambda i: (i, 0)),
            pl.BlockSpec((K, H), lambda i: (0, 0)),
            pl.BlockSpec((1, H), lambda i: (0, 0)),
        ],
        out_specs=pl.BlockSpec((A, H), lambda i: (i, 0)),
        out_shape=jax.ShapeDtypeStruct((N, H), jnp.float32),
    )(x, wT, b.reshape(1, H))


def _depth0(nfa3, nfb3, fa, wla, wlb, wn1, wn2, bias2):
    A = 400

    def body(nfa_ref, nfb_ref, fa_ref, wla_ref, wlb_ref, wn1_ref, wn2_ref, b_ref, o_ref):
        blei = b_ref[0:1, :]
        nl = jnp.zeros((A, H), jnp.float32)
        for k in range(NB):
            ya = jnp.dot(nfa_ref[k].astype(jnp.bfloat16), wla_ref[...],
                         preferred_element_type=jnp.float32)
            yb = jnp.dot(nfb_ref[k].astype(jnp.bfloat16), wlb_ref[...],
                         preferred_element_type=jnp.float32)
            nl = nl + jnp.maximum(ya + yb + blei, 0.0)
        o_ref[...] = jnp.maximum(
            jnp.dot(fa_ref[...], wn1_ref[...], preferred_element_type=jnp.float32)
            + jnp.dot(nl, wn2_ref[...], preferred_element_type=jnp.float32)
            + b_ref[1:2, :],
            0.0,
        )

    return pl.pallas_call(
        body,
        grid=(N // A,),
        in_specs=[
            pl.BlockSpec((NB, A, AF), lambda i: (0, i, 0)),
            pl.BlockSpec((NB, A, BF), lambda i: (0, i, 0)),
            pl.BlockSpec((A, H), lambda i: (i, 0)),
            pl.BlockSpec((AF, H), lambda i: (0, 0)),
            pl.BlockSpec((BF, H), lambda i: (0, 0)),
            pl.BlockSpec((H, H), lambda i: (0, 0)),
            pl.BlockSpec((H, H), lambda i: (0, 0)),
            pl.BlockSpec((2, H), lambda i: (0, 0)),
        ],
        out_specs=pl.BlockSpec((A, H), lambda i: (i, 0)),
        out_shape=jax.ShapeDtypeStruct((N, H), jnp.float32),
    )(nfa3, nfb3, fa, wla, wlb, wn1, wn2, bias2)


def _depth1(nfa3, nfb3, fa, wa2, wb2, w01, w02, bias4):
    A = 400

    def body(nfa_ref, nfb_ref, fa_ref, wa_ref, wb_ref, w01_ref, w02_ref, b_ref, o_ref):
        ba0 = b_ref[0:1, :]
        bb0 = b_ref[1:2, :]
        bg = b_ref[2:3, :]
        b02 = b_ref[3:4, :]
        fa = fa_ref[...]
        gs = jnp.dot(fa, w01_ref[...], preferred_element_type=jnp.float32) + bg
        f_nei = jnp.zeros((A, H), jnp.float32)
        for k in range(NB):
            ya = jnp.dot(nfa_ref[k].astype(jnp.bfloat16), wa_ref[...],
                         preferred_element_type=jnp.float32)
            yb = jnp.dot(nfb_ref[k].astype(jnp.bfloat16), wb_ref[...],
                         preferred_element_type=jnp.float32)
            g = jax.nn.sigmoid(ya[:, H:] + yb[:, H:] + gs) * 10.0
            f_nei = f_nei + g * (ya[:, :H] + ba0) * (yb[:, :H] + bb0)
        fs = jnp.dot(fa, w02_ref[...], preferred_element_type=jnp.float32) + b02
        o_ref[...] = f_nei * fs

    return pl.pallas_call(
        body,
        grid=(N // A,),
        in_specs=[
            pl.BlockSpec((NB, A, AF), lambda i: (0, i, 0)),
            pl.BlockSpec((NB, A, BF), lambda i: (0, i, 0)),
            pl.BlockSpec((A, H), lambda i: (i, 0)),
            pl.BlockSpec((AF, 2 * H), lambda i: (0, 0)),
            pl.BlockSpec((BF, 2 * H), lambda i: (0, 0)),
            pl.BlockSpec((H, H), lambda i: (0, 0)),
            pl.BlockSpec((H, H), lambda i: (0, 0)),
            pl.BlockSpec((4, H), lambda i: (0, 0)),
        ],
        out_specs=pl.BlockSpec((A, H), lambda i: (i, 0)),
        out_shape=jax.ShapeDtypeStruct((N, H), jnp.float32),
    )(nfa3, nfb3, fa, wa2, wb2, w01, w02, bias4)


def _readout(ah, wo0T, bo0, wo1T, bo1, wo2T, bo2):
    S = N // M

    def mean_body(x_ref, o_ref):
        o_ref[...] = jnp.sum(x_ref[...], axis=0, keepdims=True) * (1.0 / S)

    mol = pl.pallas_call(
        mean_body,
        grid=(M,),
        in_specs=[pl.BlockSpec((S, H), lambda i: (i, 0))],
        out_specs=pl.BlockSpec((1, H), lambda i: (i, 0)),
        out_shape=jax.ShapeDtypeStruct((M, H), jnp.float32),
    )(ah)

    def mlp_body(x_ref, w0, b0, w1, b1, w2, b2, o_ref):
        h = jnp.maximum(
            jnp.dot(x_ref[...], w0[...], preferred_element_type=jnp.float32) + b0[...],
            0.0,
        )
        h = jnp.maximum(
            jnp.dot(h, w1[...], preferred_element_type=jnp.float32) + b1[...], 0.0
        )
        o_ref[...] = jnp.dot(h, w2[...], preferred_element_type=jnp.float32) + b2[...]

    out = pl.pallas_call(
        mlp_body,
        in_specs=[
            pl.BlockSpec((M, H), lambda: (0, 0)),
            pl.BlockSpec((H, H), lambda: (0, 0)),
            pl.BlockSpec((1, H), lambda: (0, 0)),
            pl.BlockSpec((H, H), lambda: (0, 0)),
            pl.BlockSpec((1, H), lambda: (0, 0)),
            pl.BlockSpec((H, 1), lambda: (0, 0)),
            pl.BlockSpec((1, 1), lambda: (0, 0)),
        ],
        out_specs=pl.BlockSpec((M, 1), lambda: (0, 0)),
        out_shape=jax.ShapeDtypeStruct((M, 1), jnp.float32),
    )(mol, wo0T, bo0.reshape(1, H), wo1T, bo1.reshape(1, H), wo2T, bo2.reshape(1, 1))
    return out.reshape(-1)


def kernel(atom_features, f_bonds, a2b, a2a, a_scope, W00, b00, W01, b01, W02, b02,
           Wa0, ba0, Wb0, bb0, Wa1, ba1, Wb1, bb1, Wlei, blei, Wnew, bnew,
           Wo0, bo0, Wo1, bo1, Wo2, bo2):
    a2a_k = a2a.astype(jnp.int32).T.reshape(-1)
    a2b_k = a2b.astype(jnp.int32).T.reshape(-1)

    wla = Wlei[:, :AF].T.astype(jnp.bfloat16)
    wlb = Wlei[:, AF:].T.astype(jnp.bfloat16)
    wa2 = jnp.concatenate([Wa0.T, Wa1.T], axis=1).astype(jnp.bfloat16)
    wb2 = jnp.concatenate([Wb0.T, Wb1.T], axis=1).astype(jnp.bfloat16)
    wn1 = Wnew.T[:H]
    wn2 = Wnew.T[H:]
    bias_d0 = jnp.stack([blei, bnew])
    bias_d1 = jnp.stack([ba0, bb0, ba1 + bb1 + b01, b02])

    f_atoms = _lin(atom_features, W00.T, b00)

    nfb3 = _sc_gather(f_bonds, a2b_k, BF, jnp.float32).reshape(NB, N, BF)

    nfa3 = _sc_gather(f_atoms, a2a_k, AF, jnp.float32).reshape(NB, N, AF)
    f_atoms = _depth0(nfa3, nfb3, f_atoms, wla, wlb, wn1, wn2, bias_d0)

    nfa3 = _sc_gather(f_atoms, a2a_k, AF, jnp.float32).reshape(NB, N, AF)
    ah = _depth1(nfa3, nfb3, f_atoms, wa2, wb2, W01.T, W02.T, bias_d1)

    return _readout(ah, Wo0.T, bo0, Wo1.T, bo1, Wo2.T, bo2)


# speedup vs baseline: 2.1513x; 2.1513x over previous
import functools

import jax
import jax.numpy as jnp
from jax import lax
from jax.experimental import pallas as pl
from jax.experimental.pallas import tpu as pltpu
from jax.experimental.pallas import tpu_sc as plsc

N = 10000
NB = 16
AF = 256
BF = 16
H = 256
M = 250

_EDGES = N * NB
_NW = 32
_PER_W = _EDGES // _NW
_CH = 128
_NFULL = _PER_W // _CH
_TAIL = _PER_W - _NFULL * _CH


def _sc_gather(table, idx, d, dtype):
    mesh = plsc.VectorSubcoreMesh(core_axis_name="c", subcore_axis_name="s")

    @functools.partial(
        pl.kernel,
        mesh=mesh,
        out_type=jax.ShapeDtypeStruct((_EDGES, d), dtype),
        scratch_types=[
            pltpu.VMEM((_CH,), jnp.int32),
            pltpu.VMEM((_CH, d), dtype),
            pltpu.VMEM((_TAIL,), jnp.int32),
            pltpu.VMEM((_TAIL, d), dtype),
            pltpu.SemaphoreType.DMA,
        ],
        compiler_params=pltpu.CompilerParams(use_tc_tiling_on_sc=False),
    )
    def gather_kernel(table_hbm, idx_hbm, out_hbm, idx_v, rows_v, idx_t, rows_t, sem):
        wid = lax.axis_index("s") * 2 + lax.axis_index("c")
        base = wid * _PER_W

        def body(c, carry):
            off = base + c * _CH
            pltpu.sync_copy(idx_hbm.at[pl.ds(off, _CH)], idx_v)
            pltpu.async_copy(table_hbm.at[idx_v], rows_v, sem).wait()
            pltpu.sync_copy(rows_v, out_hbm.at[pl.ds(off, _CH)])
            return carry

        lax.fori_loop(0, _NFULL, body, 0)
        off = base + _NFULL * _CH
        pltpu.sync_copy(idx_hbm.at[pl.ds(off, _TAIL)], idx_t)
        pltpu.async_copy(table_hbm.at[idx_t], rows_t, sem).wait()
        pltpu.sync_copy(rows_t, out_hbm.at[pl.ds(off, _TAIL)])

    return gather_kernel(table, idx)


def _lin(x, wT, b):
    A = 2000
    K = x.shape[1]

    def body(x_ref, w_ref, b_ref, o_ref):
        o_ref[...] = (
            jnp.dot(x_ref[...], w_ref[...], preferred_element_type=jnp.float32)
            + b_ref[...]
        )

    return pl.pallas_call(
        body,
        grid=(N // A,),
        in_specs=[
            pl.BlockSpec((A, K), lambda i: (i, 0)),
            pl.BlockSpec((K, H), lambda i: (0, 0)),
            pl.BlockSpec((1, H), lambda i: (0, 0)),
        ],
        out_specs=pl.BlockSpec((A, H), lambda i: (i, 0)),
        out_shape=jax.ShapeDtypeStruct((N, H), jnp.float32),
    )(x, wT, b.reshape(1, H))


def _depth0(nfa3, nfb3, fa, wla, wlb, wn1, wn2, bias2):
    A = 400

    def body(nfa_ref, nfb_ref, fa_ref, wla_ref, wlb_ref, wn1_ref, wn2_ref, b_ref, o_ref):
        blei = b_ref[0:1, :]
        nl = jnp.zeros((A, H), jnp.float32)
        for k in range(NB):
            ya = jnp.dot(nfa_ref[k].astype(jnp.bfloat16), wla_ref[...],
                         preferred_element_type=jnp.float32)
            yb = jnp.dot(nfb_ref[k].astype(jnp.bfloat16), wlb_ref[...],
                         preferred_element_type=jnp.float32)
            nl = nl + jnp.maximum(ya + yb + blei, 0.0)
        o_ref[...] = jnp.maximum(
            jnp.dot(fa_ref[...], wn1_ref[...], preferred_element_type=jnp.float32)
            + jnp.dot(nl, wn2_ref[...], preferred_element_type=jnp.float32)
            + b_ref[1:2, :],
            0.0,
        )

    return pl.pallas_call(
        body,
        grid=(N // A,),
        in_specs=[
            pl.BlockSpec((NB, A, AF), lambda i: (0, i, 0)),
            pl.BlockSpec((NB, A, BF), lambda i: (0, i, 0)),
            pl.BlockSpec((A, H), lambda i: (i, 0)),
            pl.BlockSpec((AF, H), lambda i: (0, 0)),
            pl.BlockSpec((BF, H), lambda i: (0, 0)),
            pl.BlockSpec((H, H), lambda i: (0, 0)),
            pl.BlockSpec((H, H), lambda i: (0, 0)),
            pl.BlockSpec((2, H), lambda i: (0, 0)),
        ],
        out_specs=pl.BlockSpec((A, H), lambda i: (i, 0)),
        out_shape=jax.ShapeDtypeStruct((N, H), jnp.float32),
    )(nfa3, nfb3, fa, wla, wlb, wn1, wn2, bias2)


def _depth1(nfa3, nfb3, fa, wa2, wb2, w01, w02, bias4):
    A = 400

    def body(nfa_ref, nfb_ref, fa_ref, wa_ref, wb_ref, w01_ref, w02_ref, b_ref, o_ref):
        ba0 = b_ref[0:1, :]
        bb0 = b_ref[1:2, :]
        bg = b_ref[2:3, :]
        b02 = b_ref[3:4, :]
        fa = fa_ref[...]
        gs = jnp.dot(fa, w01_ref[...], preferred_element_type=jnp.float32) + bg
        f_nei = jnp.zeros((A, H), jnp.float32)
        for k in range(NB):
            ya = jnp.dot(nfa_ref[k].astype(jnp.bfloat16), wa_ref[...],
                         preferred_element_type=jnp.float32)
            yb = jnp.dot(nfb_ref[k].astype(jnp.bfloat16), wb_ref[...],
                         preferred_element_type=jnp.float32)
            g = jax.nn.sigmoid(ya[:, H:] + yb[:, H:] + gs) * 10.0
            f_nei = f_nei + g * (ya[:, :H] + ba0) * (yb[:, :H] + bb0)
        fs = jnp.dot(fa, w02_ref[...], preferred_element_type=jnp.float32) + b02
        o_ref[...] = f_nei * fs

    return pl.pallas_call(
        body,
        grid=(N // A,),
        in_specs=[
            pl.BlockSpec((NB, A, AF), lambda i: (0, i, 0)),
            pl.BlockSpec((NB, A, BF), lambda i: (0, i, 0)),
            pl.BlockSpec((A, H), lambda i: (i, 0)),
            pl.BlockSpec((AF, 2 * H), lambda i: (0, 0)),
            pl.BlockSpec((BF, 2 * H), lambda i: (0, 0)),
            pl.BlockSpec((H, H), lambda i: (0, 0)),
            pl.BlockSpec((H, H), lambda i: (0, 0)),
            pl.BlockSpec((4, H), lambda i: (0, 0)),
        ],
        out_specs=pl.BlockSpec((A, H), lambda i: (i, 0)),
        out_shape=jax.ShapeDtypeStruct((N, H), jnp.float32),
    )(nfa3, nfb3, fa, wa2, wb2, w01, w02, bias4)


def _readout(ah, wo0T, bo0, wo1T, bo1, wo2T, bo2):
    S = N // M

    def body(x_ref, w0, b0, w1, b1, w2, b2, o_ref):
        mol = jnp.sum(x_ref[...], axis=1) * (1.0 / S)
        h = jnp.maximum(
            jnp.dot(mol, w0[...], preferred_element_type=jnp.float32) + b0[...], 0.0
        )
        h = jnp.maximum(
            jnp.dot(h, w1[...], preferred_element_type=jnp.float32) + b1[...], 0.0
        )
        o_ref[...] = jnp.dot(h, w2[...], preferred_element_type=jnp.float32) + b2[...]

    out = pl.pallas_call(
        body,
        in_specs=[
            pl.BlockSpec((M, S, H), lambda: (0, 0, 0)),
            pl.BlockSpec((H, H), lambda: (0, 0)),
            pl.BlockSpec((1, H), lambda: (0, 0)),
            pl.BlockSpec((H, H), lambda: (0, 0)),
            pl.BlockSpec((1, H), lambda: (0, 0)),
            pl.BlockSpec((H, 1), lambda: (0, 0)),
            pl.BlockSpec((1, 1), lambda: (0, 0)),
        ],
        out_specs=pl.BlockSpec((M, 1), lambda: (0, 0)),
        out_shape=jax.ShapeDtypeStruct((M, 1), jnp.float32),
    )(ah.reshape(M, S, H), wo0T, bo0.reshape(1, H), wo1T, bo1.reshape(1, H),
      wo2T, bo2.reshape(1, 1))
    return out.reshape(-1)


def kernel(atom_features, f_bonds, a2b, a2a, a_scope, W00, b00, W01, b01, W02, b02,
           Wa0, ba0, Wb0, bb0, Wa1, ba1, Wb1, bb1, Wlei, blei, Wnew, bnew,
           Wo0, bo0, Wo1, bo1, Wo2, bo2):
    a2a_k = a2a.astype(jnp.int32).T.reshape(-1)
    a2b_k = a2b.astype(jnp.int32).T.reshape(-1)

    wla = Wlei[:, :AF].T.astype(jnp.bfloat16)
    wlb = Wlei[:, AF:].T.astype(jnp.bfloat16)
    wa2 = jnp.concatenate([Wa0.T, Wa1.T], axis=1).astype(jnp.bfloat16)
    wb2 = jnp.concatenate([Wb0.T, Wb1.T], axis=1).astype(jnp.bfloat16)
    wn1 = Wnew.T[:H]
    wn2 = Wnew.T[H:]
    bias_d0 = jnp.stack([blei, bnew])
    bias_d1 = jnp.stack([ba0, bb0, ba1 + bb1 + b01, b02])

    f_atoms = _lin(atom_features, W00.T, b00)

    nfb3 = _sc_gather(f_bonds, a2b_k, BF, jnp.float32).reshape(NB, N, BF)

    nfa3 = _sc_gather(f_atoms, a2a_k, AF, jnp.float32).reshape(NB, N, AF)
    f_atoms = _depth0(nfa3, nfb3, f_atoms, wla, wlb, wn1, wn2, bias_d0)

    nfa3 = _sc_gather(f_atoms, a2a_k, AF, jnp.float32).reshape(NB, N, AF)
    ah = _depth1(nfa3, nfb3, f_atoms, wa2, wb2, W01.T, W02.T, bias_d1)

    return _readout(ah, Wo0.T, bo0, Wo1.T, bo1, Wo2.T, bo2)


# speedup vs baseline: 2.2110x; 1.0277x over previous
import functools

import jax
import jax.numpy as jnp
from jax import lax
from jax.experimental import pallas as pl
from jax.experimental.pallas import tpu as pltpu
from jax.experimental.pallas import tpu_sc as plsc

N = 10000
NB = 16
AF = 256
BF = 16
H = 256
M = 250

_EDGES = N * NB
_NW = 32
_PER_W = _EDGES // _NW
_CH = 128
_NFULL = _PER_W // _CH
_TAIL = _PER_W - _NFULL * _CH


def _sc_gather(table, idx, d, dtype):
    mesh = plsc.VectorSubcoreMesh(core_axis_name="c", subcore_axis_name="s")

    @functools.partial(
        pl.kernel,
        mesh=mesh,
        out_type=jax.ShapeDtypeStruct((_EDGES, d), dtype),
        scratch_types=[
            pltpu.VMEM((_CH,), jnp.int32),
            pltpu.VMEM((_CH, d), dtype),
            pltpu.VMEM((_TAIL,), jnp.int32),
            pltpu.VMEM((_TAIL, d), dtype),
            pltpu.SemaphoreType.DMA,
        ],
        compiler_params=pltpu.CompilerParams(use_tc_tiling_on_sc=False),
    )
    def gather_kernel(table_hbm, idx_hbm, out_hbm, idx_v, rows_v, idx_t, rows_t, sem):
        wid = lax.axis_index("s") * 2 + lax.axis_index("c")
        base = wid * _PER_W

        def body(c, carry):
            off = base + c * _CH
            pltpu.sync_copy(idx_hbm.at[pl.ds(off, _CH)], idx_v)
            pltpu.async_copy(table_hbm.at[idx_v], rows_v, sem).wait()
            pltpu.sync_copy(rows_v, out_hbm.at[pl.ds(off, _CH)])
            return carry

        lax.fori_loop(0, _NFULL, body, 0)
        off = base + _NFULL * _CH
        pltpu.sync_copy(idx_hbm.at[pl.ds(off, _TAIL)], idx_t)
        pltpu.async_copy(table_hbm.at[idx_t], rows_t, sem).wait()
        pltpu.sync_copy(rows_t, out_hbm.at[pl.ds(off, _TAIL)])

    return gather_kernel(table, idx)


def _lin(x, wT, b):
    A = 2000
    K = x.shape[1]

    def body(x_ref, w_ref, b_ref, o_ref):
        o_ref[...] = (
            jnp.dot(x_ref[...], w_ref[...], preferred_element_type=jnp.float32)
            + b_ref[...]
        )

    return pl.pallas_call(
        body,
        grid=(N // A,),
        in_specs=[
            pl.BlockSpec((A, K), lambda i: (i, 0)),
            pl.BlockSpec((K, H), lambda i: (0, 0)),
            pl.BlockSpec((1, H), lambda i: (0, 0)),
        ],
        out_specs=pl.BlockSpec((A, H), lambda i: (i, 0)),
        out_shape=jax.ShapeDtypeStruct((N, H), jnp.float32),
    )(x, wT, b.reshape(1, H))


def _depth0(nfa3, nfb3, fa, wla, wlb, wn1, wn2, bias2):
    A = 400

    def body(nfa_ref, nfb_ref, fa_ref, wla_ref, wlb_ref, wn1_ref, wn2_ref, b_ref, o_ref):
        blei = b_ref[0:1, :]
        nl = jnp.zeros((A, H), jnp.float32)
        for k in range(NB):
            ya = jnp.dot(nfa_ref[k], wla_ref[...],
                         preferred_element_type=jnp.float32)
            yb = jnp.dot(nfb_ref[k].astype(jnp.bfloat16), wlb_ref[...],
                         preferred_element_type=jnp.float32)
            nl = nl + jnp.maximum(ya + yb + blei, 0.0)
        o_ref[...] = jnp.maximum(
            jnp.dot(fa_ref[...], wn1_ref[...], preferred_element_type=jnp.float32)
            + jnp.dot(nl, wn2_ref[...], preferred_element_type=jnp.float32)
            + b_ref[1:2, :],
            0.0,
        )

    return pl.pallas_call(
        body,
        grid=(N // A,),
        in_specs=[
            pl.BlockSpec((NB, A, AF), lambda i: (0, i, 0)),
            pl.BlockSpec((NB, A, BF), lambda i: (0, i, 0)),
            pl.BlockSpec((A, H), lambda i: (i, 0)),
            pl.BlockSpec((AF, H), lambda i: (0, 0)),
            pl.BlockSpec((BF, H), lambda i: (0, 0)),
            pl.BlockSpec((H, H), lambda i: (0, 0)),
            pl.BlockSpec((H, H), lambda i: (0, 0)),
            pl.BlockSpec((2, H), lambda i: (0, 0)),
        ],
        out_specs=pl.BlockSpec((A, H), lambda i: (i, 0)),
        out_shape=jax.ShapeDtypeStruct((N, H), jnp.float32),
    )(nfa3, nfb3, fa, wla, wlb, wn1, wn2, bias2)


def _depth1(nfa3, nfb3, fa, wa2, wb2, w01, w02, bias4):
    A = 400

    def body(nfa_ref, nfb_ref, fa_ref, wa_ref, wb_ref, w01_ref, w02_ref, b_ref, o_ref):
        ba0 = b_ref[0:1, :]
        bb0 = b_ref[1:2, :]
        bg = b_ref[2:3, :]
        b02 = b_ref[3:4, :]
        fa = fa_ref[...]
        gs = jnp.dot(fa, w01_ref[...], preferred_element_type=jnp.float32) + bg
        f_nei = jnp.zeros((A, H), jnp.float32)
        for k in range(NB):
            ya = jnp.dot(nfa_ref[k].astype(jnp.bfloat16), wa_ref[...],
                         preferred_element_type=jnp.float32)
            yb = jnp.dot(nfb_ref[k].astype(jnp.bfloat16), wb_ref[...],
                         preferred_element_type=jnp.float32)
            g = jax.nn.sigmoid(ya[:, H:] + yb[:, H:] + gs) * 10.0
            f_nei = f_nei + g * (ya[:, :H] + ba0) * (yb[:, :H] + bb0)
        fs = jnp.dot(fa, w02_ref[...], preferred_element_type=jnp.float32) + b02
        o_ref[...] = f_nei * fs

    return pl.pallas_call(
        body,
        grid=(N // A,),
        in_specs=[
            pl.BlockSpec((NB, A, AF), lambda i: (0, i, 0)),
            pl.BlockSpec((NB, A, BF), lambda i: (0, i, 0)),
            pl.BlockSpec((A, H), lambda i: (i, 0)),
            pl.BlockSpec((AF, 2 * H), lambda i: (0, 0)),
            pl.BlockSpec((BF, 2 * H), lambda i: (0, 0)),
            pl.BlockSpec((H, H), lambda i: (0, 0)),
            pl.BlockSpec((H, H), lambda i: (0, 0)),
            pl.BlockSpec((4, H), lambda i: (0, 0)),
        ],
        out_specs=pl.BlockSpec((A, H), lambda i: (i, 0)),
        out_shape=jax.ShapeDtypeStruct((N, H), jnp.float32),
    )(nfa3, nfb3, fa, wa2, wb2, w01, w02, bias4)


def _readout(ah, wo0T, bo0, wo1T, bo1, wo2T, bo2):
    S = N // M

    def body(x_ref, w0, b0, w1, b1, w2, b2, o_ref):
        mol = jnp.sum(x_ref[...], axis=1) * (1.0 / S)
        h = jnp.maximum(
            jnp.dot(mol, w0[...], preferred_element_type=jnp.float32) + b0[...], 0.0
        )
        h = jnp.maximum(
            jnp.dot(h, w1[...], preferred_element_type=jnp.float32) + b1[...], 0.0
        )
        o_ref[...] = jnp.dot(h, w2[...], preferred_element_type=jnp.float32) + b2[...]

    out = pl.pallas_call(
        body,
        in_specs=[
            pl.BlockSpec((M, S, H), lambda: (0, 0, 0)),
            pl.BlockSpec((H, H), lambda: (0, 0)),
            pl.BlockSpec((1, H), lambda: (0, 0)),
            pl.BlockSpec((H, H), lambda: (0, 0)),
            pl.BlockSpec((1, H), lambda: (0, 0)),
            pl.BlockSpec((H, 1), lambda: (0, 0)),
            pl.BlockSpec((1, 1), lambda: (0, 0)),
        ],
        out_specs=pl.BlockSpec((M, 1), lambda: (0, 0)),
        out_shape=jax.ShapeDtypeStruct((M, 1), jnp.float32),
    )(ah.reshape(M, S, H), wo0T, bo0.reshape(1, H), wo1T, bo1.reshape(1, H),
      wo2T, bo2.reshape(1, 1))
    return out.reshape(-1)


def kernel(atom_features, f_bonds, a2b, a2a, a_scope, W00, b00, W01, b01, W02, b02,
           Wa0, ba0, Wb0, bb0, Wa1, ba1, Wb1, bb1, Wlei, blei, Wnew, bnew,
           Wo0, bo0, Wo1, bo1, Wo2, bo2):
    a2a_k = a2a.astype(jnp.int32).T.reshape(-1)
    a2b_k = a2b.astype(jnp.int32).T.reshape(-1)

    wla = Wlei[:, :AF].T.astype(jnp.bfloat16)
    wlb = Wlei[:, AF:].T.astype(jnp.bfloat16)
    wa2 = jnp.concatenate([Wa0.T, Wa1.T], axis=1).astype(jnp.bfloat16)
    wb2 = jnp.concatenate([Wb0.T, Wb1.T], axis=1).astype(jnp.bfloat16)
    wn1 = Wnew.T[:H]
    wn2 = Wnew.T[H:]
    bias_d0 = jnp.stack([blei, bnew])
    bias_d1 = jnp.stack([ba0, bb0, ba1 + bb1 + b01, b02])

    f_atoms = _lin(atom_features, W00.T, b00)

    nfb3 = _sc_gather(f_bonds, a2b_k, BF, jnp.float32).reshape(NB, N, BF)

    nfa3 = _sc_gather(f_atoms.astype(jnp.bfloat16), a2a_k, AF,
                      jnp.bfloat16).reshape(NB, N, AF)
    f_atoms = _depth0(nfa3, nfb3, f_atoms, wla, wlb, wn1, wn2, bias_d0)

    nfa3 = _sc_gather(f_atoms.astype(jnp.bfloat16), a2a_k, AF,
                      jnp.bfloat16).reshape(NB, N, AF)
    ah = _depth1(nfa3, nfb3, f_atoms, wa2, wb2, W01.T, W02.T, bias_d1)

    return _readout(ah, Wo0.T, bo0, Wo1.T, bo1, Wo2.T, bo2)


# speedup vs baseline: 2.2907x; 1.0360x over previous
import functools

import jax
import jax.numpy as jnp
from jax import lax
from jax.experimental import pallas as pl
from jax.experimental.pallas import tpu as pltpu
from jax.experimental.pallas import tpu_sc as plsc

N = 10000
NB = 16
AF = 256
BF = 16
H = 256
M = 250

_EDGES = N * NB
_NW = 32
_CH = 128
_NCH = _EDGES // _CH
_FULL = _NCH // _NW
_EXTRA = _NCH - _FULL * _NW


def _sc_gather(table, idx2, d, dtype):
    mesh = plsc.VectorSubcoreMesh(core_axis_name="c", subcore_axis_name="s")

    @functools.partial(
        pl.kernel,
        mesh=mesh,
        out_type=jax.ShapeDtypeStruct((_EDGES, d), dtype),
        scratch_types=[
            pltpu.VMEM((_CH,), jnp.int32),
            pltpu.VMEM((_CH,), jnp.int32),
            pltpu.VMEM((_CH,), jnp.int32),
            pltpu.VMEM((_CH, d), dtype),
            pltpu.VMEM((_CH, d), dtype),
            pltpu.VMEM((_CH, d), dtype),
            pltpu.SemaphoreType.DMA,
            pltpu.SemaphoreType.DMA,
            pltpu.SemaphoreType.DMA,
            pltpu.SemaphoreType.DMA,
            pltpu.SemaphoreType.DMA,
            pltpu.SemaphoreType.DMA,
        ],
        compiler_params=pltpu.CompilerParams(use_tc_tiling_on_sc=False),
    )
    def gather_kernel(table_hbm, idx_hbm, out_hbm,
                      i0, i1, i2, r0, r1, r2, g0, g1, g2, w0, w1, w2):
        idx_v = (i0, i1, i2)
        rows = (r0, r1, r2)
        gsem = (g0, g1, g2)
        wsem = (w0, w1, w2)
        wid = lax.axis_index("s") * 2 + lax.axis_index("c")

        def chunk_of(t):
            return wid + _NW * t

        def load_idx(b, t):
            pltpu.sync_copy(idx_hbm.at[chunk_of(t)], idx_v[b])

        def fire_gather(b):
            pltpu.async_copy(table_hbm.at[idx_v[b]], rows[b], gsem[b])

        def wait_gather(b):
            pltpu.make_async_copy(table_hbm.at[idx_v[b]], rows[b], gsem[b]).wait()

        def fire_wb(b, t):
            pltpu.async_copy(rows[b], out_hbm.at[pl.ds(chunk_of(t) * _CH, _CH)],
                             wsem[b])

        def wait_wb(b, t):
            pltpu.make_async_copy(rows[b],
                                  out_hbm.at[pl.ds(chunk_of(t) * _CH, _CH)],
                                  wsem[b]).wait()

        for b in range(3):
            load_idx(b, b)
        fire_gather(0)

        def body(j, carry):
            for b in range(3):
                t = 3 * j + b
                wait_gather(b)
                fire_wb(b, t)

                @pl.when(jnp.logical_or(
                    t + 3 < _FULL,
                    jnp.logical_and(t + 3 == _FULL, wid < _EXTRA)))
                def _():
                    load_idx(b, t + 3)

                b1 = (b + 1) % 3

                @pl.when(t >= 2)
                def _():
                    wait_wb(b1, t - 2)

                @pl.when(t + 1 < _FULL)
                def _():
                    fire_gather(b1)
            return carry

        lax.fori_loop(0, _FULL // 3, body, 0)

        wait_wb((_FULL - 2) % 3, _FULL - 2)
        wait_wb((_FULL - 1) % 3, _FULL - 1)

        @pl.when(wid < _EXTRA)
        def _():
            fire_gather(0)
            wait_gather(0)
            fire_wb(0, _FULL)
            wait_wb(0, _FULL)

    return gather_kernel(table, idx2)


def _lin(x, wT, b):
    A = 2000
    K = x.shape[1]

    def body(x_ref, w_ref, b_ref, o_ref):
        o_ref[...] = (
            jnp.dot(x_ref[...], w_ref[...], preferred_element_type=jnp.float32)
            + b_ref[...]
        )

    return pl.pallas_call(
        body,
        grid=(N // A,),
        in_specs=[
            pl.BlockSpec((A, K), lambda i: (i, 0)),
            pl.BlockSpec((K, H), lambda i: (0, 0)),
            pl.BlockSpec((1, H), lambda i: (0, 0)),
        ],
        out_specs=pl.BlockSpec((A, H), lambda i: (i, 0)),
        out_shape=jax.ShapeDtypeStruct((N, H), jnp.float32),
    )(x, wT, b.reshape(1, H))


def _depth0(nfa3, nfb3, fa, wla, wlb, wn1, wn2, bias2):
    A = 400

    def body(nfa_ref, nfb_ref, fa_ref, wla_ref, wlb_ref, wn1_ref, wn2_ref, b_ref, o_ref):
        blei = b_ref[0:1, :]
        nl = jnp.zeros((A, H), jnp.float32)
        for k in range(NB):
            ya = jnp.dot(nfa_ref[k], wla_ref[...],
                         preferred_element_type=jnp.float32)
            yb = jnp.dot(nfb_ref[k].astype(jnp.bfloat16), wlb_ref[...],
                         preferred_element_type=jnp.float32)
            nl = nl + jnp.maximum(ya + yb + blei, 0.0)
        o_ref[...] = jnp.maximum(
            jnp.dot(fa_ref[...], wn1_ref[...], preferred_element_type=jnp.float32)
            + jnp.dot(nl, wn2_ref[...], preferred_element_type=jnp.float32)
            + b_ref[1:2, :],
            0.0,
        )

    return pl.pallas_call(
        body,
        grid=(N // A,),
        in_specs=[
            pl.BlockSpec((NB, A, AF), lambda i: (0, i, 0)),
            pl.BlockSpec((NB, A, BF), lambda i: (0, i, 0)),
            pl.BlockSpec((A, H), lambda i: (i, 0)),
            pl.BlockSpec((AF, H), lambda i: (0, 0)),
            pl.BlockSpec((BF, H), lambda i: (0, 0)),
            pl.BlockSpec((H, H), lambda i: (0, 0)),
            pl.BlockSpec((H, H), lambda i: (0, 0)),
            pl.BlockSpec((2, H), lambda i: (0, 0)),
        ],
        out_specs=pl.BlockSpec((A, H), lambda i: (i, 0)),
        out_shape=jax.ShapeDtypeStruct((N, H), jnp.float32),
    )(nfa3, nfb3, fa, wla, wlb, wn1, wn2, bias2)


def _depth1(nfa3, nfb3, fa, wa2, wb2, w01, w02, bias4):
    A = 400

    def body(nfa_ref, nfb_ref, fa_ref, wa_ref, wb_ref, w01_ref, w02_ref, b_ref, o_ref):
        ba0 = b_ref[0:1, :]
        bb0 = b_ref[1:2, :]
        bg = b_ref[2:3, :]
        b02 = b_ref[3:4, :]
        fa = fa_ref[...]
        gs = jnp.dot(fa, w01_ref[...], preferred_element_type=jnp.float32) + bg
        f_nei = jnp.zeros((A, H), jnp.float32)
        for k in range(NB):
            ya = jnp.dot(nfa_ref[k].astype(jnp.bfloat16), wa_ref[...],
                         preferred_element_type=jnp.float32)
            yb = jnp.dot(nfb_ref[k].astype(jnp.bfloat16), wb_ref[...],
                         preferred_element_type=jnp.float32)
            g = jax.nn.sigmoid(ya[:, H:] + yb[:, H:] + gs) * 10.0
            f_nei = f_nei + g * (ya[:, :H] + ba0) * (yb[:, :H] + bb0)
        fs = jnp.dot(fa, w02_ref[...], preferred_element_type=jnp.float32) + b02
        o_ref[...] = f_nei * fs

    return pl.pallas_call(
        body,
        grid=(N // A,),
        in_specs=[
            pl.BlockSpec((NB, A, AF), lambda i: (0, i, 0)),
            pl.BlockSpec((NB, A, BF), lambda i: (0, i, 0)),
            pl.BlockSpec((A, H), lambda i: (i, 0)),
            pl.BlockSpec((AF, 2 * H), lambda i: (0, 0)),
            pl.BlockSpec((BF, 2 * H), lambda i: (0, 0)),
            pl.BlockSpec((H, H), lambda i: (0, 0)),
            pl.BlockSpec((H, H), lambda i: (0, 0)),
            pl.BlockSpec((4, H), lambda i: (0, 0)),
        ],
        out_specs=pl.BlockSpec((A, H), lambda i: (i, 0)),
        out_shape=jax.ShapeDtypeStruct((N, H), jnp.float32),
    )(nfa3, nfb3, fa, wa2, wb2, w01, w02, bias4)


def _readout(ah, wo0T, bo0, wo1T, bo1, wo2T, bo2):
    S = N // M

    def body(x_ref, w0, b0, w1, b1, w2, b2, o_ref):
        mol = jnp.sum(x_ref[...], axis=1) * (1.0 / S)
        h = jnp.maximum(
            jnp.dot(mol, w0[...], preferred_element_type=jnp.float32) + b0[...], 0.0
        )
        h = jnp.maximum(
            jnp.dot(h, w1[...], preferred_element_type=jnp.float32) + b1[...], 0.0
        )
        o_ref[...] = jnp.dot(h, w2[...], preferred_element_type=jnp.float32) + b2[...]

    out = pl.pallas_call(
        body,
        in_specs=[
            pl.BlockSpec((M, S, H), lambda: (0, 0, 0)),
            pl.BlockSpec((H, H), lambda: (0, 0)),
            pl.BlockSpec((1, H), lambda: (0, 0)),
            pl.BlockSpec((H, H), lambda: (0, 0)),
            pl.BlockSpec((1, H), lambda: (0, 0)),
            pl.BlockSpec((H, 1), lambda: (0, 0)),
            pl.BlockSpec((1, 1), lambda: (0, 0)),
        ],
        out_specs=pl.BlockSpec((M, 1), lambda: (0, 0)),
        out_shape=jax.ShapeDtypeStruct((M, 1), jnp.float32),
    )(ah.reshape(M, S, H), wo0T, bo0.reshape(1, H), wo1T, bo1.reshape(1, H),
      wo2T, bo2.reshape(1, 1))
    return out.reshape(-1)


def kernel(atom_features, f_bonds, a2b, a2a, a_scope, W00, b00, W01, b01, W02, b02,
           Wa0, ba0, Wb0, bb0, Wa1, ba1, Wb1, bb1, Wlei, blei, Wnew, bnew,
           Wo0, bo0, Wo1, bo1, Wo2, bo2):
    a2a_k = a2a.astype(jnp.int32).T.reshape(_NCH, _CH)
    a2b_k = a2b.astype(jnp.int32).T.reshape(_NCH, _CH)

    wla = Wlei[:, :AF].T.astype(jnp.bfloat16)
    wlb = Wlei[:, AF:].T.astype(jnp.bfloat16)
    wa2 = jnp.concatenate([Wa0.T, Wa1.T], axis=1).astype(jnp.bfloat16)
    wb2 = jnp.concatenate([Wb0.T, Wb1.T], axis=1).astype(jnp.bfloat16)
    wn1 = Wnew.T[:H]
    wn2 = Wnew.T[H:]
    bias_d0 = jnp.stack([blei, bnew])
    bias_d1 = jnp.stack([ba0, bb0, ba1 + bb1 + b01, b02])

    f_atoms = _lin(atom_features, W00.T, b00)

    nfb3 = _sc_gather(f_bonds, a2b_k, BF, jnp.float32).reshape(NB, N, BF)

    nfa3 = _sc_gather(f_atoms.astype(jnp.bfloat16), a2a_k, AF,
                      jnp.bfloat16).reshape(NB, N, AF)
    f_atoms = _depth0(nfa3, nfb3, f_atoms, wla, wlb, wn1, wn2, bias_d0)

    nfa3 = _sc_gather(f_atoms.astype(jnp.bfloat16), a2a_k, AF,
                      jnp.bfloat16).reshape(NB, N, AF)
    ah = _depth1(nfa3, nfb3, f_atoms, wa2, wb2, W01.T, W02.T, bias_d1)

    return _readout(ah, Wo0.T, bo0, Wo1.T, bo1, Wo2.T, bo2)


# speedup vs baseline: 3.1373x; 1.3696x over previous
import functools

import jax
import jax.numpy as jnp
from jax import lax
from jax.experimental import pallas as pl
from jax.experimental.pallas import tpu as pltpu
from jax.experimental.pallas import tpu_sc as plsc

N = 10000
NB = 16
AF = 256
BF = 16
H = 256
M = 250

_EDGES = N * NB
_NW = 32
_CH = 128
_NCH = _EDGES // _CH
_FULL = _NCH // _NW
_EXTRA = _NCH - _FULL * _NW


def _sc_gather(table, idx2, d, dtype, tiled):
    mesh = plsc.VectorSubcoreMesh(core_axis_name="c", subcore_axis_name="s")

    @functools.partial(
        pl.kernel,
        mesh=mesh,
        out_type=jax.ShapeDtypeStruct((_EDGES, d), dtype),
        scratch_types=[
            pltpu.VMEM((_CH,), jnp.int32),
            pltpu.VMEM((_CH,), jnp.int32),
            pltpu.VMEM((_CH,), jnp.int32),
            pltpu.VMEM((_CH, d), dtype),
            pltpu.VMEM((_CH, d), dtype),
            pltpu.VMEM((_CH, d), dtype),
            pltpu.SemaphoreType.DMA,
            pltpu.SemaphoreType.DMA,
            pltpu.SemaphoreType.DMA,
            pltpu.SemaphoreType.DMA,
            pltpu.SemaphoreType.DMA,
            pltpu.SemaphoreType.DMA,
        ],
        compiler_params=pltpu.CompilerParams(use_tc_tiling_on_sc=tiled),
    )
    def gather_kernel(table_hbm, idx_hbm, out_hbm,
                      i0, i1, i2, r0, r1, r2, g0, g1, g2, w0, w1, w2):
        idx_v = (i0, i1, i2)
        rows = (r0, r1, r2)
        gsem = (g0, g1, g2)
        wsem = (w0, w1, w2)
        wid = lax.axis_index("s") * 2 + lax.axis_index("c")

        def chunk_of(t):
            return wid + _NW * t

        def load_idx(b, t):
            pltpu.sync_copy(idx_hbm.at[chunk_of(t)], idx_v[b])

        def fire_gather(b):
            pltpu.async_copy(table_hbm.at[idx_v[b]], rows[b], gsem[b])

        def wait_gather(b):
            pltpu.make_async_copy(table_hbm.at[idx_v[b]], rows[b], gsem[b]).wait()

        def fire_wb(b, t):
            pltpu.async_copy(rows[b], out_hbm.at[pl.ds(chunk_of(t) * _CH, _CH)],
                             wsem[b])

        def wait_wb(b, t):
            pltpu.make_async_copy(rows[b],
                                  out_hbm.at[pl.ds(chunk_of(t) * _CH, _CH)],
                                  wsem[b]).wait()

        for b in range(3):
            load_idx(b, b)
        fire_gather(0)

        def body(j, carry):
            for b in range(3):
                t = 3 * j + b
                wait_gather(b)
                fire_wb(b, t)

                @pl.when(jnp.logical_or(
                    t + 3 < _FULL,
                    jnp.logical_and(t + 3 == _FULL, wid < _EXTRA)))
                def _():
                    load_idx(b, t + 3)

                b1 = (b + 1) % 3

                @pl.when(t >= 2)
                def _():
                    wait_wb(b1, t - 2)

                @pl.when(t + 1 < _FULL)
                def _():
                    fire_gather(b1)
            return carry

        lax.fori_loop(0, _FULL // 3, body, 0)

        wait_wb((_FULL - 2) % 3, _FULL - 2)
        wait_wb((_FULL - 1) % 3, _FULL - 1)

        @pl.when(wid < _EXTRA)
        def _():
            fire_gather(0)
            wait_gather(0)
            fire_wb(0, _FULL)
            wait_wb(0, _FULL)

    return gather_kernel(table, idx2)


def _lin(x, wT, b):
    A = 2000
    K = x.shape[1]

    def body(x_ref, w_ref, b_ref, o_ref):
        o_ref[...] = (
            jnp.dot(x_ref[...], w_ref[...], preferred_element_type=jnp.float32)
            + b_ref[...]
        )

    return pl.pallas_call(
        body,
        grid=(N // A,),
        in_specs=[
            pl.BlockSpec((A, K), lambda i: (i, 0)),
            pl.BlockSpec((K, H), lambda i: (0, 0)),
            pl.BlockSpec((1, H), lambda i: (0, 0)),
        ],
        out_specs=pl.BlockSpec((A, H), lambda i: (i, 0)),
        out_shape=jax.ShapeDtypeStruct((N, H), jnp.float32),
    )(x, wT, b.reshape(1, H))


_A = 200
_R = _A * NB


def _depth0(nfa, nfb, fa, wla, wlb, wn1, wn2, bias2):
    def body(nfa_ref, nfb_ref, fa_ref, wla_ref, wlb_ref, wn1_ref, wn2_ref, b_ref, o_ref):
        blei = b_ref[0:1, :]
        ya = jnp.dot(nfa_ref[...].astype(jnp.bfloat16), wla_ref[...],
                     preferred_element_type=jnp.float32)
        yb = jnp.dot(nfb_ref[...].astype(jnp.bfloat16), wlb_ref[...],
                     preferred_element_type=jnp.float32)
        t = jnp.maximum(ya + yb + blei, 0.0)
        nl = jnp.sum(t.reshape(_A, NB, H), axis=1)
        o_ref[...] = jnp.maximum(
            jnp.dot(fa_ref[...], wn1_ref[...], preferred_element_type=jnp.float32)
            + jnp.dot(nl, wn2_ref[...], preferred_element_type=jnp.float32)
            + b_ref[1:2, :],
            0.0,
        )

    return pl.pallas_call(
        body,
        grid=(N // _A,),
        in_specs=[
            pl.BlockSpec((_R, AF), lambda i: (i, 0)),
            pl.BlockSpec((_R, BF), lambda i: (i, 0)),
            pl.BlockSpec((_A, H), lambda i: (i, 0)),
            pl.BlockSpec((AF, H), lambda i: (0, 0)),
            pl.BlockSpec((BF, H), lambda i: (0, 0)),
            pl.BlockSpec((H, H), lambda i: (0, 0)),
            pl.BlockSpec((H, H), lambda i: (0, 0)),
            pl.BlockSpec((2, H), lambda i: (0, 0)),
        ],
        out_specs=pl.BlockSpec((_A, H), lambda i: (i, 0)),
        out_shape=jax.ShapeDtypeStruct((N, H), jnp.float32),
    )(nfa, nfb, fa, wla, wlb, wn1, wn2, bias2)


def _depth1(nfa, nfb, fa, wa2, wb2, w01, w02, bias4):
    def body(nfa_ref, nfb_ref, fa_ref, wa_ref, wb_ref, w01_ref, w02_ref, b_ref, o_ref):
        ba0 = b_ref[0:1, :]
        bb0 = b_ref[1:2, :]
        bg = b_ref[2:3, :]
        b02 = b_ref[3:4, :]
        fa = fa_ref[...]
        gs = jnp.dot(fa, w01_ref[...], preferred_element_type=jnp.float32) + bg
        ya = jnp.dot(nfa_ref[...].astype(jnp.bfloat16), wa_ref[...],
                     preferred_element_type=jnp.float32)
        yb = jnp.dot(nfb_ref[...].astype(jnp.bfloat16), wb_ref[...],
                     preferred_element_type=jnp.float32)
        ya3 = ya.reshape(_A, NB, 2 * H)
        yb3 = yb.reshape(_A, NB, 2 * H)
        g = jax.nn.sigmoid(ya3[:, :, H:] + yb3[:, :, H:] + gs[:, None, :]) * 10.0
        f_nei = jnp.sum(g * (ya3[:, :, :H] + ba0) * (yb3[:, :, :H] + bb0), axis=1)
        fs = jnp.dot(fa, w02_ref[...], preferred_element_type=jnp.float32) + b02
        o_ref[...] = f_nei * fs

    return pl.pallas_call(
        body,
        grid=(N // _A,),
        in_specs=[
            pl.BlockSpec((_R, AF), lambda i: (i, 0)),
            pl.BlockSpec((_R, BF), lambda i: (i, 0)),
            pl.BlockSpec((_A, H), lambda i: (i, 0)),
            pl.BlockSpec((AF, 2 * H), lambda i: (0, 0)),
            pl.BlockSpec((BF, 2 * H), lambda i: (0, 0)),
            pl.BlockSpec((H, H), lambda i: (0, 0)),
            pl.BlockSpec((H, H), lambda i: (0, 0)),
            pl.BlockSpec((4, H), lambda i: (0, 0)),
        ],
        out_specs=pl.BlockSpec((_A, H), lambda i: (i, 0)),
        out_shape=jax.ShapeDtypeStruct((N, H), jnp.float32),
    )(nfa, nfb, fa, wa2, wb2, w01, w02, bias4)


def _readout(ah, wo0T, bo0, wo1T, bo1, wo2T, bo2):
    S = N // M

    def body(x_ref, w0, b0, w1, b1, w2, b2, o_ref):
        mol = jnp.sum(x_ref[...], axis=1) * (1.0 / S)
        h = jnp.maximum(
            jnp.dot(mol, w0[...], preferred_element_type=jnp.float32) + b0[...], 0.0
        )
        h = jnp.maximum(
            jnp.dot(h, w1[...], preferred_element_type=jnp.float32) + b1[...], 0.0
        )
        o_ref[...] = jnp.dot(h, w2[...], preferred_element_type=jnp.float32) + b2[...]

    out = pl.pallas_call(
        body,
        in_specs=[
            pl.BlockSpec((M, S, H), lambda: (0, 0, 0)),
            pl.BlockSpec((H, H), lambda: (0, 0)),
            pl.BlockSpec((1, H), lambda: (0, 0)),
            pl.BlockSpec((H, H), lambda: (0, 0)),
            pl.BlockSpec((1, H), lambda: (0, 0)),
            pl.BlockSpec((H, 1), lambda: (0, 0)),
            pl.BlockSpec((1, 1), lambda: (0, 0)),
        ],
        out_specs=pl.BlockSpec((M, 1), lambda: (0, 0)),
        out_shape=jax.ShapeDtypeStruct((M, 1), jnp.float32),
    )(ah.reshape(M, S, H), wo0T, bo0.reshape(1, H), wo1T, bo1.reshape(1, H),
      wo2T, bo2.reshape(1, 1))
    return out.reshape(-1)


def kernel(atom_features, f_bonds, a2b, a2a, a_scope, W00, b00, W01, b01, W02, b02,
           Wa0, ba0, Wb0, bb0, Wa1, ba1, Wb1, bb1, Wlei, blei, Wnew, bnew,
           Wo0, bo0, Wo1, bo1, Wo2, bo2):
    a2a_k = a2a.astype(jnp.int32).reshape(_NCH, _CH)
    a2b_k = a2b.astype(jnp.int32).reshape(_NCH, _CH)

    wla = Wlei[:, :AF].T.astype(jnp.bfloat16)
    wlb = Wlei[:, AF:].T.astype(jnp.bfloat16)
    wa2 = jnp.concatenate([Wa0.T, Wa1.T], axis=1).astype(jnp.bfloat16)
    wb2 = jnp.concatenate([Wb0.T, Wb1.T], axis=1).astype(jnp.bfloat16)
    wn1 = Wnew.T[:H]
    wn2 = Wnew.T[H:]
    bias_d0 = jnp.stack([blei, bnew])
    bias_d1 = jnp.stack([ba0, bb0, ba1 + bb1 + b01, b02])

    f_atoms = _lin(atom_features, W00.T, b00)

    nfb = _sc_gather(f_bonds, a2b_k, BF, jnp.float32, tiled=False)

    nfa = _sc_gather(f_atoms, a2a_k, AF, jnp.float32, tiled=True)
    f_atoms = _depth0(nfa, nfb, f_atoms, wla, wlb, wn1, wn2, bias_d0)

    nfa = _sc_gather(f_atoms, a2a_k, AF, jnp.float32, tiled=True)
    ah = _depth1(nfa, nfb, f_atoms, wa2, wb2, W01.T, W02.T, bias_d1)

    return _readout(ah, Wo0.T, bo0, Wo1.T, bo1, Wo2.T, bo2)
